# baseline (speedup 1.0000x reference)
import jax
import jax.numpy as jnp
from jax.experimental import pallas as pl


_ROWS = 8192
_COLS = 1024
_BLOCK_ROWS = 512


def _body(z_ref, o_ref):
    v = z_ref[...]
    lane = jax.lax.broadcasted_iota(jnp.int32, v.shape, dimension=1)
    even = (lane & 1) == 0
    p = jnp.where(even, jnp.roll(v, -1, axis=1), jnp.roll(v, 1, axis=1))
    x0 = jnp.where(even, v, p)
    x1 = jnp.where(even, p, v)
    a0 = jnp.abs(x0)
    a1 = jnp.abs(x1)
    ub = jnp.float32(1.5)
    inside = (a0 < ub) & (a1 < ub)
    bd = (a0 > ub) | (a1 > ub)
    pos1 = x1 > 0.0
    one = jnp.float32(1.0)
    neg0 = (pos1 & (x0 <= 0.0)) | ((~pos1) & (x0 < 0.0))
    sgn0 = jnp.where(neg0, -one, one)
    sgn1 = jnp.where(pos1, one, -one)
    qv = inside & (pos1 | (x0 != 0.0))
    s = sgn0 * sgn1
    se = jnp.where(even, one, -one)
    c05 = jnp.float32(0.05)
    aq = one - c05 * s * se
    cq = c05 * s * jnp.where(even, sgn0, -sgn1)
    yq = aq * v + cq
    ybd = jnp.float32(0.9505) * v + jnp.where(even, jnp.float32(-0.02),
                                              jnp.float32(0.02)) * p
    zero = jnp.float32(0.0)
    o_ref[...] = jnp.where(bd, ybd, jnp.where(qv, yq, zero))


def kernel(z):
    n = z.shape[0]
    zf = z.reshape(_ROWS, _COLS)
    out = pl.pallas_call(
        _body,
        out_shape=jax.ShapeDtypeStruct((_ROWS, _COLS), jnp.float32),
        grid=(_ROWS // _BLOCK_ROWS,),
        in_specs=[pl.BlockSpec((_BLOCK_ROWS, _COLS), lambda i: (i, 0))],
        out_specs=pl.BlockSpec((_BLOCK_ROWS, _COLS), lambda i: (i, 0)),
    )(zf)
    return out.reshape(n, 2)


# speedup vs baseline: 1.0026x; 1.0026x over previous
import jax
import jax.numpy as jnp
from jax.experimental import pallas as pl


_ROWS = 65536
_COLS = 128
_BLOCK_ROWS = 4096


def _body(z_ref, o_ref):
    v = z_ref[...]
    lane = jax.lax.broadcasted_iota(jnp.int32, v.shape, dimension=1)
    even = (lane & 1) == 0
    p = jnp.where(even, jnp.roll(v, -1, axis=1), jnp.roll(v, 1, axis=1))
    x0 = jnp.where(even, v, p)
    x1 = jnp.where(even, p, v)
    a0 = jnp.abs(x0)
    a1 = jnp.abs(x1)
    ub = jnp.float32(1.5)
    inside = (a0 < ub) & (a1 < ub)
    bd = (a0 > ub) | (a1 > ub)
    pos1 = x1 > 0.0
    one = jnp.float32(1.0)
    neg0 = (pos1 & (x0 <= 0.0)) | ((~pos1) & (x0 < 0.0))
    sgn0 = jnp.where(neg0, -one, one)
    sgn1 = jnp.where(pos1, one, -one)
    qv = inside & (pos1 | (x0 != 0.0))
    s = sgn0 * sgn1
    se = jnp.where(even, one, -one)
    c05 = jnp.float32(0.05)
    aq = one - c05 * s * se
    cq = c05 * s * jnp.where(even, sgn0, -sgn1)
    yq = aq * v + cq
    ybd = jnp.float32(0.9505) * v + jnp.where(even, jnp.float32(-0.02),
                                              jnp.float32(0.02)) * p
    zero = jnp.float32(0.0)
    o_ref[...] = jnp.where(bd, ybd, jnp.where(qv, yq, zero))


def kernel(z):
    n = z.shape[0]
    zf = z.reshape(_ROWS, _COLS)
    out = pl.pallas_call(
        _body,
        out_shape=jax.ShapeDtypeStruct((_ROWS, _COLS), jnp.float32),
        grid=(_ROWS // _BLOCK_ROWS,),
        in_specs=[pl.BlockSpec((_BLOCK_ROWS, _COLS), lambda i: (i, 0))],
        out_specs=pl.BlockSpec((_BLOCK_ROWS, _COLS), lambda i: (i, 0)),
    )(zf)
    return out.reshape(n, 2)


# speedup vs baseline: 79.2173x; 79.0149x over previous
import jax
import jax.numpy as jnp
from jax.experimental import pallas as pl

_N = 4194304
_ROWS = _N // 128
_BLOCK_ROWS = 2048


def _body(x0_ref, x1_ref, y0_ref, y1_ref):
    x0 = x0_ref[...]
    x1 = x1_ref[...]
    a0 = jnp.abs(x0)
    a1 = jnp.abs(x1)
    ub = jnp.float32(1.5)
    inside = (a0 < ub) & (a1 < ub)
    bd = (a0 > ub) | (a1 > ub)
    pos1 = x1 > 0.0
    one = jnp.float32(1.0)
    neg0 = (pos1 & (x0 <= 0.0)) | ((~pos1) & (x0 < 0.0))
    sgn0 = jnp.where(neg0, -one, one)
    sgn1 = jnp.where(pos1, one, -one)
    qv = inside & (pos1 | (x0 != 0.0))
    s = sgn0 * sgn1
    c05 = jnp.float32(0.05)
    cs = c05 * s
    y0q = (one - cs) * x0 + cs * sgn0
    y1q = (one + cs) * x1 - cs * sgn1
    c9505 = jnp.float32(0.9505)
    c02 = jnp.float32(0.02)
    y0b = c9505 * x0 - c02 * x1
    y1b = c02 * x0 + c9505 * x1
    zero = jnp.float32(0.0)
    y0_ref[...] = jnp.where(bd, y0b, jnp.where(qv, y0q, zero))
    y1_ref[...] = jnp.where(bd, y1b, jnp.where(qv, y1q, zero))


def kernel(z):
    x0 = z[:, 0].reshape(_ROWS, 128)
    x1 = z[:, 1].reshape(_ROWS, 128)
    spec = pl.BlockSpec((_BLOCK_ROWS, 128), lambda i: (i, 0))
    y0, y1 = pl.pallas_call(
        _body,
        out_shape=(jax.ShapeDtypeStruct((_ROWS, 128), jnp.float32),
                   jax.ShapeDtypeStruct((_ROWS, 128), jnp.float32)),
        grid=(_ROWS // _BLOCK_ROWS,),
        in_specs=[spec, spec],
        out_specs=(spec, spec),
    )(x0, x1)
    return jnp.stack([y0.reshape(_N), y1.reshape(_N)], axis=1)


# speedup vs baseline: 209.5568x; 2.6453x over previous
import jax
import jax.numpy as jnp
from jax.experimental import pallas as pl

_N = 4194304
_GROUPS = _N // 512
_BLOCK_GROUPS = 512


def _body(u_ref, o_ref):
    v = u_ref[...]
    sub = jax.lax.broadcasted_iota(jnp.int32, v.shape, dimension=1)
    ev = (sub & 1) == 0
    p = jnp.where(ev, jnp.roll(v, -1, axis=1), jnp.roll(v, 1, axis=1))
    x0 = jnp.where(ev, v, p)
    x1 = jnp.where(ev, p, v)
    a0 = jnp.abs(x0)
    a1 = jnp.abs(x1)
    ub = jnp.float32(1.5)
    inside = (a0 < ub) & (a1 < ub)
    bd = (a0 > ub) | (a1 > ub)
    pos1 = x1 > 0.0
    one = jnp.float32(1.0)
    neg0 = (pos1 & (x0 <= 0.0)) | ((~pos1) & (x0 < 0.0))
    sgn0 = jnp.where(neg0, -one, one)
    sgn1 = jnp.where(pos1, one, -one)
    qv = inside & (pos1 | (x0 != 0.0))
    s = sgn0 * sgn1
    cs = jnp.float32(0.05) * s
    se = jnp.where(ev, one, -one)
    aq = one - cs * se
    cq = cs * jnp.where(ev, sgn0, -sgn1)
    yq = aq * v + cq
    ybd = jnp.float32(0.9505) * v + jnp.where(ev, jnp.float32(-0.02),
                                              jnp.float32(0.02)) * p
    zero = jnp.float32(0.0)
    o_ref[...] = jnp.where(bd, ybd, jnp.where(qv, yq, zero))


def kernel(z):
    u = (z.reshape(_GROUPS, 4, 128, 2)
         .transpose(0, 1, 3, 2)
         .reshape(_GROUPS, 8, 128))
    spec = pl.BlockSpec((_BLOCK_GROUPS, 8, 128), lambda i: (i, 0, 0))
    y = pl.pallas_call(
        _body,
        out_shape=jax.ShapeDtypeStruct((_GROUPS, 8, 128), jnp.float32),
        grid=(_GROUPS // _BLOCK_GROUPS,),
        in_specs=[spec],
        out_specs=spec,
    )(u)
    return (y.reshape(_GROUPS, 4, 2, 128)
            .transpose(0, 1, 3, 2)
            .reshape(_N, 2))


# speedup vs baseline: 227.3818x; 1.0851x over previous
import jax
import jax.numpy as jnp
from jax.experimental import pallas as pl

_N = 4194304
_GROUPS = _N // 512
_BLOCK_GROUPS = 512


def _body(u_ref, o_ref):
    v = u_ref[...]
    sub = jax.lax.broadcasted_iota(jnp.int32, v.shape, dimension=1)
    ev = (sub & 1) == 0
    p = jnp.where(ev, jnp.roll(v, -1, axis=1), jnp.roll(v, 1, axis=1))
    x0 = jnp.where(ev, v, p)
    x1 = jnp.where(ev, p, v)
    m = jnp.maximum(jnp.abs(v), jnp.abs(p))
    ub = jnp.float32(1.5)
    inside = m < ub
    bd = m > ub
    pos1 = x1 > 0.0
    one = jnp.float32(1.0)
    c05 = jnp.float32(0.05)
    neg0 = (pos1 & (x0 <= 0.0)) | ((~pos1) & (x0 < 0.0))
    sgn0 = jnp.where(neg0, -one, one)
    h = jnp.where(pos1, c05, -c05)
    cs = sgn0 * h
    qv = inside & (pos1 | (x0 != 0.0))
    se = jnp.where(ev, one, -one)
    aq = one - cs * se
    cq = jnp.where(ev, h, jnp.where(neg0, c05, -c05))
    yq = aq * v + cq
    ybd = jnp.float32(0.9505) * v + jnp.where(ev, jnp.float32(-0.02),
                                              jnp.float32(0.02)) * p
    zero = jnp.float32(0.0)
    o_ref[...] = jnp.where(bd, ybd, jnp.where(qv, yq, zero))


def kernel(z):
    u = (z.reshape(_GROUPS, 4, 128, 2)
         .transpose(0, 1, 3, 2)
         .reshape(_GROUPS, 8, 128))
    spec = pl.BlockSpec((_BLOCK_GROUPS, 8, 128), lambda i: (i, 0, 0))
    y = pl.pallas_call(
        _body,
        out_shape=jax.ShapeDtypeStruct((_GROUPS, 8, 128), jnp.float32),
        grid=(_GROUPS // _BLOCK_GROUPS,),
        in_specs=[spec],
        out_specs=spec,
    )(u)
    return (y.reshape(_GROUPS, 4, 2, 128)
            .transpose(0, 1, 3, 2)
            .reshape(_N, 2))


# speedup vs baseline: 245.0822x; 1.0778x over previous
import jax
import jax.numpy as jnp
from jax.experimental import pallas as pl

_N = 4194304
_GROUPS = _N // 512
_BLOCK_GROUPS = 512


def _body(u_ref, o_ref):
    v = u_ref[...]
    sub = jax.lax.broadcasted_iota(jnp.int32, v.shape, dimension=1)
    ev = (sub & 1) == 0
    ge = jnp.where(ev, jnp.float32(0.05), jnp.float32(-0.05))
    nge = jnp.where(ev, jnp.float32(-0.05), jnp.float32(0.05))
    cb = jnp.where(ev, jnp.float32(-0.02), jnp.float32(0.02))
    p = jnp.where(ev, jnp.roll(v, -1, axis=1), jnp.roll(v, 1, axis=1))
    g = jnp.where(p > 0.0, ge, nge)
    av = jnp.abs(v)
    ap = jnp.abs(p)
    yq = v - g * (av - jnp.float32(1.0))
    m = jnp.maximum(av, ap)
    ub = jnp.float32(1.5)
    inside = m < ub
    bd = m > ub
    ybd = jnp.float32(0.9505) * v + cb * p
    good = jnp.where(ev, jnp.maximum(av, p), jnp.maximum(ap, v)) > 0.0
    y = jnp.where(inside & good, yq, jnp.float32(0.0))
    o_ref[...] = jnp.where(bd, ybd, y)


def kernel(z):
    u = (z.reshape(_GROUPS, 4, 128, 2)
         .transpose(0, 1, 3, 2)
         .reshape(_GROUPS, 8, 128))
    spec = pl.BlockSpec((_BLOCK_GROUPS, 8, 128), lambda i: (i, 0, 0))
    y = pl.pallas_call(
        _body,
        out_shape=jax.ShapeDtypeStruct((_GROUPS, 8, 128), jnp.float32),
        grid=(_GROUPS // _BLOCK_GROUPS,),
        in_specs=[spec],
        out_specs=spec,
    )(u)
    return (y.reshape(_GROUPS, 4, 2, 128)
            .transpose(0, 1, 3, 2)
            .reshape(_N, 2))
